# baseline (speedup 1.0000x reference)
import functools

import jax
import jax.numpy as jnp
from jax import lax
from jax.experimental import pallas as pl
from jax.experimental.pallas import tpu as pltpu
from jax.experimental.pallas import tpu_sc as plsc

_B, _C, _D = 1024, 128, 256
_R, _RC = 8, 16
_NC, _NS = 2, 16
_NW = _NC * _NS
_BPW = _B // _NW


def _sc_body(x_hbm, *refs):
    outs = refs[:_R]
    sem = refs[_R]
    wid = lax.axis_index("s") * _NC + lax.axis_index("c")
    base = wid * _BPW
    copies = [
        pltpu.async_copy(
            x_hbm.at[pl.ds(base, _BPW), pl.ds(k * _RC, _RC), :],
            outs[k].at[pl.ds(base, _BPW)],
            sem,
        )
        for k in range(_R)
    ]
    for c in copies:
        c.wait()


_sc_call = pl.kernel(
    _sc_body,
    out_type=tuple(
        jax.ShapeDtypeStruct((_B, _RC, _D), jnp.float32) for _ in range(_R)
    ),
    mesh=plsc.VectorSubcoreMesh(core_axis_name="c", subcore_axis_name="s"),
    scratch_types=[pltpu.SemaphoreType.DMA],
)


@jax.jit
def kernel(x):
    return _sc_call(x)


# speedup vs baseline: 35.5808x; 35.5808x over previous
import jax
import jax.numpy as jnp
from jax import lax
from jax.experimental import pallas as pl
from jax.experimental.pallas import tpu as pltpu
from jax.experimental.pallas import tpu_sc as plsc

_B, _C, _D = 1024, 128, 256
_R, _RC = 8, 16
_NC, _NS = 2, 16
_NW = _NC * _NS
_BPW = _B // _NW
_NBUF = 3


def _sc_body(x_hbm, *refs):
    outs = refs[:_R]
    buf = refs[_R]
    in_sem = refs[_R + 1]
    out_sem = refs[_R + 2]
    wid = lax.axis_index("s") * _NC + lax.axis_index("c")
    base = wid * _BPW

    def start_in(i):
        return pltpu.async_copy(x_hbm.at[base + i], buf.at[i % _NBUF], in_sem)

    def start_outs(i):
        return [
            pltpu.async_copy(
                buf.at[i % _NBUF, pl.ds(k * _RC, _RC)],
                outs[k].at[base + i],
                out_sem,
            )
            for k in range(_R)
        ]

    in_copies = {0: start_in(0), 1: start_in(1)}
    pending = {}
    for i in range(_BPW):
        in_copies.pop(i).wait()
        if i - 1 in pending:
            for c in pending.pop(i - 1):
                c.wait()
        if i + 2 < _BPW:
            in_copies[i + 2] = start_in(i + 2)
        pending[i] = start_outs(i)
    for cs in pending.values():
        for c in cs:
            c.wait()


_sc_call = pl.kernel(
    _sc_body,
    out_type=tuple(
        jax.ShapeDtypeStruct((_B, _RC, _D), jnp.float32) for _ in range(_R)
    ),
    mesh=plsc.VectorSubcoreMesh(core_axis_name="c", subcore_axis_name="s"),
    scratch_types=[
        pltpu.VMEM((_NBUF, _C, _D), jnp.float32),
        pltpu.SemaphoreType.DMA,
        pltpu.SemaphoreType.DMA,
    ],
)


@jax.jit
def kernel(x):
    return _sc_call(x)


# speedup vs baseline: 35.9949x; 1.0116x over previous
import jax
import jax.numpy as jnp
from jax import lax
from jax.experimental import pallas as pl
from jax.experimental.pallas import tpu as pltpu
from jax.experimental.pallas import tpu_sc as plsc

_B, _C, _D = 1024, 128, 256
_R, _RC = 8, 16
_NC, _NS = 2, 16
_NW = _NC * _NS
_BPW = _B // _NW
_BCH = 8
_NCH = _BPW // _BCH
_NBUF = 3


def _sc_body(x_hbm, *refs):
    outs = refs[:_R]
    buf = refs[_R]
    in_sem = refs[_R + 1]
    out_sem = refs[_R + 2]
    wid = lax.axis_index("s") * _NC + lax.axis_index("c")
    base = wid * _BPW

    tiles = [(k, j) for k in range(_R) for j in range(_NCH)]
    n = len(tiles)

    def start_in(i):
        k, j = tiles[i]
        return pltpu.async_copy(
            x_hbm.at[pl.ds(base + j * _BCH, _BCH), pl.ds(k * _RC, _RC)],
            buf.at[i % _NBUF],
            in_sem,
        )

    def start_out(i):
        k, j = tiles[i]
        return pltpu.async_copy(
            buf.at[i % _NBUF],
            outs[k].at[pl.ds(base + j * _BCH, _BCH)],
            out_sem,
        )

    in_copies = {0: start_in(0), 1: start_in(1)}
    pending = {}
    for i in range(n):
        in_copies.pop(i).wait()
        if i - 1 in pending:
            pending.pop(i - 1).wait()
        if i + 2 < n:
            in_copies[i + 2] = start_in(i + 2)
        pending[i] = start_out(i)
    for c in pending.values():
        c.wait()


_sc_call = pl.kernel(
    _sc_body,
    out_type=tuple(
        jax.ShapeDtypeStruct((_B, _RC, _D), jnp.float32) for _ in range(_R)
    ),
    mesh=plsc.VectorSubcoreMesh(core_axis_name="c", subcore_axis_name="s"),
    scratch_types=[
        pltpu.VMEM((_NBUF, _BCH, _RC, _D), jnp.float32),
        pltpu.SemaphoreType.DMA,
        pltpu.SemaphoreType.DMA,
    ],
)


@jax.jit
def kernel(x):
    return _sc_call(x)
